# baseline (speedup 1.0000x reference)
import functools

import jax
import jax.numpy as jnp
from jax import lax
from jax.experimental import pallas as pl
from jax.experimental.pallas import tpu as pltpu
from jax.experimental.pallas import tpu_sc as plsc

B, S, D = 4, 4096, 1024
N_TOK = B * S
PAD = 1
EPS = 1e-5

NC, NS, L = 2, 16, 16
NW = NC * NS
TOK_PER_W = N_TOK // NW
W_PER_ROW = S // TOK_PER_W
T = 16
NCHUNK = TOK_PER_W // T
KV = D // L
MVREGS = TOK_PER_W // L

_mesh = plsc.VectorSubcoreMesh(
    core_axis_name="c", subcore_axis_name="s", num_cores=NC, num_subcores=NS
)

_DNUMS = lax.GatherDimensionNumbers(
    offset_dims=(), collapsed_slice_dims=(0,), start_index_map=(0,))


def _lane_gather(x, idx):
  return lax.gather(x, idx[:, None], _DNUMS, slice_sizes=(1,),
                    mode=lax.GatherScatterMode.PROMISE_IN_BOUNDS)


def _vsum(x, iota):
  for sh in (8, 4, 2, 1):
    x = x + _lane_gather(x, jnp.bitwise_xor(iota, sh))
  return x


def _vcumsum(x, iota, zero):
  for sh in (1, 2, 4, 8):
    g = _lane_gather(x, jnp.maximum(iota - sh, 0))
    x = x + jnp.where(iota >= sh, g, zero)
  return x


def _rsqrt(x):
  i = lax.bitcast_convert_type(x, jnp.int32)
  i = jnp.int32(0x5F3759DF) - lax.shift_right_arithmetic(i, 1)
  y = lax.bitcast_convert_type(i, jnp.float32)
  for _ in range(4):
    y = y * (1.5 - 0.5 * x * y * y)
  return y


@functools.partial(
    pl.kernel,
    out_type=jax.ShapeDtypeStruct((N_TOK, D), jnp.float32),
    mesh=_mesh,
    scratch_types=[
        pltpu.VMEM((TOK_PER_W,), jnp.int32),
        pltpu.VMEM((TOK_PER_W,), jnp.int32),
        pltpu.VMEM((S,), jnp.int32),
        pltpu.VMEM((D,), jnp.float32),
        pltpu.VMEM((D,), jnp.float32),
        pltpu.VMEM((T, D), jnp.float32),
        pltpu.VMEM((T, D), jnp.float32),
        pltpu.VMEM((T, D), jnp.float32),
        pltpu.SemaphoreType.DMA,
        pltpu.SemaphoreType.DMA,
    ],
)
def _embed_ln(inp_hbm, msk_hbm, word_hbm, pos_hbm, g_hbm, b_hbm, out_hbm,
              idx_v, pos_v, mrow, g_v, b_v, wbuf, pbuf, obuf, wsem, psem):
  wid = lax.axis_index("s") * NC + lax.axis_index("c")
  row = wid // W_PER_ROW
  chunk_in_row = wid % W_PER_ROW
  base = wid * TOK_PER_W

  pltpu.sync_copy(g_hbm, g_v)
  pltpu.sync_copy(b_hbm, b_v)
  pltpu.sync_copy(inp_hbm.at[pl.ds(base, TOK_PER_W)], idx_v)
  pltpu.sync_copy(msk_hbm.at[pl.ds(row * S, S)], mrow)

  iota = lax.iota(jnp.int32, L)
  izero = jnp.zeros((L,), jnp.int32)

  def _prefix(k, acc):
    return acc + mrow[pl.ds(k * L, L)]
  offset = lax.fori_loop(0, chunk_in_row * MVREGS, _prefix, izero)
  offset = _vsum(offset, iota)

  slab = chunk_in_row * TOK_PER_W
  for jj in range(MVREGS):
    v = mrow[pl.ds(slab + jj * L, L)]
    cs = _vcumsum(v, iota, izero)
    pos_v[pl.ds(jj * L, L)] = (cs + offset) * v + PAD
    offset = offset + _lane_gather(cs, jnp.full((L,), L - 1, jnp.int32))

  inv_d = jnp.float32(1.0 / D)

  def _chunk(i, _):
    cw = pltpu.make_async_copy(
        word_hbm.at[idx_v.at[pl.ds(i * T, T)]], wbuf, wsem)
    cp = pltpu.make_async_copy(
        pos_hbm.at[pos_v.at[pl.ds(i * T, T)]], pbuf, psem)
    cw.start()
    cp.start()
    cw.wait()
    cp.wait()

    for j in range(T):
      def _stats(k, carry):
        s, q = carry
        x = wbuf[j, pl.ds(k * L, L)] + pbuf[j, pl.ds(k * L, L)]
        return s + x, q + x * x
      s, q = lax.fori_loop(0, KV, _stats,
                           (jnp.zeros((L,), jnp.float32),
                            jnp.zeros((L,), jnp.float32)))
      mu = _vsum(s, iota) * inv_d
      var = _vsum(q, iota) * inv_d - mu * mu
      a = _rsqrt(var + EPS)

      def _norm(k, _):
        x = wbuf[j, pl.ds(k * L, L)] + pbuf[j, pl.ds(k * L, L)]
        y = (x - mu) * a
        obuf[j, pl.ds(k * L, L)] = y * g_v[pl.ds(k * L, L)] + b_v[pl.ds(k * L, L)]
        return 0
      lax.fori_loop(0, KV, _norm, 0)

    pltpu.sync_copy(obuf, out_hbm.at[pl.ds(base + i * T, T)])
    return 0

  lax.fori_loop(0, NCHUNK, _chunk, 0)


def kernel(input, mask, word_w, pos_w, ln_g, ln_b):
  inp = input.reshape(-1).astype(jnp.int32)
  msk = mask.reshape(-1).astype(jnp.int32)
  out = _embed_ln(inp, msk, word_w, pos_w, ln_g, ln_b)
  return out.reshape(B, S, D)
